# baseline (speedup 1.0000x reference)
import functools

import jax
import jax.numpy as jnp
from jax import lax
from jax.experimental import pallas as pl
from jax.experimental.pallas import tpu as pltpu
from jax.experimental.pallas import tpu_sc as plsc

_IBIG = 1 << 20
_NC, _NS, _L = 2, 16, 16
_NW = _NC * _NS
_RPC = 8
_NACC = 4
_BB = 256


def _sc_select(nop_ref, pt_ref, tbu_ref, out_ref, ptb, nopb, tbub, actb, sem,
               *, rows, n_jobs, n_mas, n_trs, n_ops):
    cid = lax.axis_index("c")
    sid = lax.axis_index("s")
    wid = sid * _NC + cid
    base = wid * rows
    n_jc = nopb.shape[1] // _L
    nchunk = rows // _RPC
    lane = lax.iota(jnp.int32, _L)

    pltpu.sync_copy(nop_ref.at[pl.ds(base, rows)], nopb)
    pltpu.sync_copy(tbu_ref.at[pl.ds(base, rows)], tbub)
    pltpu.async_copy(pt_ref.at[pl.ds(base, _RPC)], ptb.at[0], sem)
    pltpu.async_copy(pt_ref.at[pl.ds(base + _RPC, _RPC)], ptb.at[1], sem)

    def chunk_body(c, carry):
        buf = lax.rem(c, 2)
        pltpu.make_async_copy(pt_ref.at[pl.ds(base, _RPC)], ptb.at[0], sem).wait()
        bvec = jnp.full((_L,), buf, jnp.int32)

        def row_body(rr, carry2):
            for r_off in range(2):
                r = rr * 2 + r_off
                row = c * _RPC + r
                rvec = jnp.full((_L,), r, jnp.int32)
                vals = [jnp.full((_L,), jnp.inf, jnp.float32)
                        for _ in range(_NACC)]
                keys = [jnp.full((_L,), _IBIG, jnp.int32)
                        for _ in range(_NACC)]
                for jc in range(n_jc):
                    idx16 = nopb[row, pl.ds(jc * _L, _L)]
                    jkey = (jc * _L + lane) * n_mas
                    pad = n_jobs - jc * _L
                    for m in range(n_mas):
                        a = m % _NACC
                        v = plsc.load_gather(ptb, [bvec, rvec,
                                                   idx16 + m * n_ops])
                        better = (v < vals[a]) & (v != 0.0)
                        if pad < _L:
                            better = better & (lane < pad)
                        vals[a] = jnp.where(better, v, vals[a])
                        keys[a] = jnp.where(better, jkey + m, keys[a])
                vm, km = vals[0], keys[0]
                for a in range(1, _NACC):
                    take = (vals[a] < vm) | ((vals[a] == vm) & (keys[a] < km))
                    vm = jnp.where(take, vals[a], vm)
                    km = jnp.where(take, keys[a], km)
                minv = jnp.min(vm)
                fkey = jnp.min(jnp.where(vm == minv, km, _IBIG))
                fkey = jnp.where(minv == jnp.inf, 0, fkey)
                tv = tbub[row]
                tkey = jnp.min(jnp.where(tv == jnp.min(tv), lane, _L))
                act = 1 + fkey * n_trs + tkey
                actb[r] = jnp.full((_L,), act, jnp.int32)
            return carry2

        lax.fori_loop(0, _RPC // 2, row_body, 0)
        pltpu.sync_copy(actb, out_ref.at[pl.ds(base + c * _RPC, _RPC)])

        @pl.when(c + 2 < nchunk)
        def _prefetch():
            pltpu.async_copy(pt_ref.at[pl.ds(base + (c + 2) * _RPC, _RPC)],
                             ptb.at[buf], sem)

        return carry

    lax.fori_loop(0, nchunk, chunk_body, 0)


def _tc_onehot(act_ref, out_ref):
    act = act_ref[:, :1]
    n_cols = out_ref.shape[1]
    col = lax.broadcasted_iota(jnp.int32, (act_ref.shape[0], n_cols), 1)
    out_ref[...] = jnp.where(col == act, 1.0, 0.0).astype(jnp.float32)


def kernel(job_done, machine_busy_until, truck_location, next_op, proc_times,
           truck_busy_until, action_mask):
    B, n_jobs = job_done.shape
    n_mas = machine_busy_until.shape[1]
    n_trs = truck_location.shape[1]
    n_ops = proc_times.shape[2]
    n_cols = 1 + n_jobs * n_mas * n_trs
    rows = B // _NW

    jpad = (-n_jobs) % _L
    nop_p = jnp.pad(next_op, ((0, 0), (0, jpad)))
    tbu_p = jnp.pad(truck_busy_until, ((0, 0), (0, _L - n_trs)),
                    constant_values=jnp.inf)
    pt2 = proc_times.reshape(B, n_mas * n_ops)

    sel = functools.partial(_sc_select, rows=rows, n_jobs=n_jobs,
                            n_mas=n_mas, n_trs=n_trs, n_ops=n_ops)
    act16 = pl.kernel(
        sel,
        out_type=jax.ShapeDtypeStruct((B, _L), jnp.int32),
        mesh=plsc.VectorSubcoreMesh(core_axis_name="c", subcore_axis_name="s",
                                    num_cores=_NC, num_subcores=_NS),
        compiler_params=pltpu.CompilerParams(needs_layout_passes=False),
        scratch_types=[
            pltpu.VMEM((2, _RPC, n_mas * n_ops), jnp.float32),
            pltpu.VMEM((rows, n_jobs + jpad), jnp.int32),
            pltpu.VMEM((rows, _L), jnp.float32),
            pltpu.VMEM((_RPC, _L), jnp.int32),
            pltpu.SemaphoreType.DMA,
        ],
    )(nop_p, pt2, tbu_p)

    logits = pl.pallas_call(
        _tc_onehot,
        grid=(B // _BB,),
        in_specs=[pl.BlockSpec((_BB, _L), lambda i: (i, 0))],
        out_specs=pl.BlockSpec((_BB, n_cols), lambda i: (i, 0)),
        out_shape=jax.ShapeDtypeStruct((B, n_cols), jnp.float32),
    )(act16)
    return (logits, action_mask)


# speedup vs baseline: 1.0274x; 1.0274x over previous
import functools

import jax
import jax.numpy as jnp
from jax import lax
from jax.experimental import pallas as pl
from jax.experimental.pallas import tpu as pltpu
from jax.experimental.pallas import tpu_sc as plsc

_IBIG = 1 << 20
_NC, _NS, _L = 2, 16, 16
_NW = _NC * _NS
_RPC = 8
_NACC = 4
_BB = 256


def _sc_select(nop_ref, pt_ref, tbu_ref, out_ref, ptb, nopb, tbub, actb, sem,
               *, rows, n_jobs, n_mas, n_trs, n_ops):
    cid = lax.axis_index("c")
    sid = lax.axis_index("s")
    wid = sid * _NC + cid
    base = wid * rows
    n_jc = nopb.shape[1] // _L
    nchunk = rows // _RPC
    lane = lax.iota(jnp.int32, _L)

    pltpu.sync_copy(nop_ref.at[pl.ds(base, rows)], nopb)
    pltpu.sync_copy(tbu_ref.at[pl.ds(base, rows)], tbub)
    pltpu.async_copy(pt_ref.at[pl.ds(base, _RPC)], ptb.at[0], sem)
    pltpu.async_copy(pt_ref.at[pl.ds(base + _RPC, _RPC)], ptb.at[1], sem)

    def chunk_body(c, carry):
        buf = lax.rem(c, 2)
        pltpu.make_async_copy(pt_ref.at[pl.ds(base, _RPC)], ptb.at[0], sem).wait()
        bvec = jnp.full((_L,), buf, jnp.int32)

        def row_body(r, carry2):
            row = c * _RPC + r
            rvec = jnp.full((_L,), r, jnp.int32)
            umax = jnp.uint32(0xFFFFFFFF)
            uvals = [jnp.full((_L,), umax, jnp.uint32) for _ in range(_NACC)]
            keys = [jnp.full((_L,), _IBIG, jnp.int32) for _ in range(_NACC)]
            for jc in range(n_jc):
                idx16 = nopb[row, pl.ds(jc * _L, _L)]
                jkey = (jc * _L + lane) * n_mas
                pad = n_jobs - jc * _L
                for m in range(n_mas):
                    a = m % _NACC
                    v = plsc.load_gather(ptb, [bvec, rvec, idx16 + m * n_ops])
                    u = plsc.bitcast(v, jnp.uint32) - jnp.uint32(1)
                    if pad < _L:
                        u = jnp.where(lane < pad, u, umax)
                    better = u < uvals[a]
                    uvals[a] = jnp.where(better, u, uvals[a])
                    keys[a] = jnp.where(better, jkey + m, keys[a])
            vm, km = uvals[0], keys[0]
            for a in range(1, _NACC):
                take = (uvals[a] < vm) | ((uvals[a] == vm) & (keys[a] < km))
                vm = jnp.where(take, uvals[a], vm)
                km = jnp.where(take, keys[a], km)
            sm = plsc.bitcast(vm ^ jnp.uint32(0x80000000), jnp.int32)
            minv = jnp.min(sm)
            fkey = jnp.min(jnp.where(sm == minv, km, _IBIG))
            fkey = jnp.where(minv == jnp.int32(0x7FFFFFFF), 0, fkey)
            tv = tbub[row]
            tkey = jnp.min(jnp.where(tv == jnp.min(tv), lane, _L))
            act = 1 + fkey * n_trs + tkey
            actb[r] = jnp.full((_L,), act, jnp.int32)
            return carry2

        lax.fori_loop(0, _RPC, row_body, 0)
        pltpu.sync_copy(actb, out_ref.at[pl.ds(base + c * _RPC, _RPC)])

        @pl.when(c + 2 < nchunk)
        def _prefetch():
            pltpu.async_copy(pt_ref.at[pl.ds(base + (c + 2) * _RPC, _RPC)],
                             ptb.at[buf], sem)

        return carry

    lax.fori_loop(0, nchunk, chunk_body, 0)


def _tc_onehot(act_ref, out_ref):
    act = act_ref[:, :1]
    n_cols = out_ref.shape[1]
    col = lax.broadcasted_iota(jnp.int32, (act_ref.shape[0], n_cols), 1)
    out_ref[...] = jnp.where(col == act, 1.0, 0.0).astype(jnp.float32)


def kernel(job_done, machine_busy_until, truck_location, next_op, proc_times,
           truck_busy_until, action_mask):
    B, n_jobs = job_done.shape
    n_mas = machine_busy_until.shape[1]
    n_trs = truck_location.shape[1]
    n_ops = proc_times.shape[2]
    n_cols = 1 + n_jobs * n_mas * n_trs
    rows = B // _NW

    jpad = (-n_jobs) % _L
    nop_p = jnp.pad(next_op, ((0, 0), (0, jpad)))
    tbu_p = jnp.pad(truck_busy_until, ((0, 0), (0, _L - n_trs)),
                    constant_values=jnp.inf)
    pt2 = proc_times.reshape(B, n_mas * n_ops)

    sel = functools.partial(_sc_select, rows=rows, n_jobs=n_jobs,
                            n_mas=n_mas, n_trs=n_trs, n_ops=n_ops)
    act16 = pl.kernel(
        sel,
        out_type=jax.ShapeDtypeStruct((B, _L), jnp.int32),
        mesh=plsc.VectorSubcoreMesh(core_axis_name="c", subcore_axis_name="s",
                                    num_cores=_NC, num_subcores=_NS),
        compiler_params=pltpu.CompilerParams(needs_layout_passes=False),
        scratch_types=[
            pltpu.VMEM((2, _RPC, n_mas * n_ops), jnp.float32),
            pltpu.VMEM((rows, n_jobs + jpad), jnp.int32),
            pltpu.VMEM((rows, _L), jnp.float32),
            pltpu.VMEM((_RPC, _L), jnp.int32),
            pltpu.SemaphoreType.DMA,
        ],
    )(nop_p, pt2, tbu_p)

    logits = pl.pallas_call(
        _tc_onehot,
        grid=(B // _BB,),
        in_specs=[pl.BlockSpec((_BB, _L), lambda i: (i, 0))],
        out_specs=pl.BlockSpec((_BB, n_cols), lambda i: (i, 0)),
        out_shape=jax.ShapeDtypeStruct((B, n_cols), jnp.float32),
    )(act16)
    return (logits, action_mask)


# speedup vs baseline: 1.0289x; 1.0015x over previous
import functools

import jax
import jax.numpy as jnp
from jax import lax
from jax.experimental import pallas as pl
from jax.experimental.pallas import tpu as pltpu
from jax.experimental.pallas import tpu_sc as plsc

_IBIG = 1 << 20
_NC, _NS, _L = 2, 16, 16
_NW = _NC * _NS
_RPC = 8
_NACC = 8
_BB = 256


def _sc_select(nop_ref, pt_ref, tbu_ref, out_ref, ptb, nopb, tbub, actb, sem,
               *, rows, n_jobs, n_mas, n_trs, n_ops):
    cid = lax.axis_index("c")
    sid = lax.axis_index("s")
    wid = sid * _NC + cid
    base = wid * rows
    n_jc = nopb.shape[1] // _L
    nchunk = rows // _RPC
    lane = lax.iota(jnp.int32, _L)

    pltpu.sync_copy(nop_ref.at[pl.ds(base, rows)], nopb)
    pltpu.sync_copy(tbu_ref.at[pl.ds(base, rows)], tbub)
    pltpu.async_copy(pt_ref.at[pl.ds(base, _RPC)], ptb.at[0], sem)
    pltpu.async_copy(pt_ref.at[pl.ds(base + _RPC, _RPC)], ptb.at[1], sem)

    def chunk_body(c, carry):
        buf = lax.rem(c, 2)
        pltpu.make_async_copy(pt_ref.at[pl.ds(base, _RPC)], ptb.at[0], sem).wait()
        bvec = jnp.full((_L,), buf, jnp.int32)

        def row_body(r, carry2):
            row = c * _RPC + r
            rvec = jnp.full((_L,), r, jnp.int32)
            umax = jnp.uint32(0xFFFFFFFF)
            uvals = [jnp.full((_L,), umax, jnp.uint32) for _ in range(_NACC)]
            keys = [jnp.full((_L,), _IBIG, jnp.int32) for _ in range(_NACC)]
            for jc in range(n_jc):
                idx16 = nopb[row, pl.ds(jc * _L, _L)]
                jkey = (jc * _L + lane) * n_mas
                pad = n_jobs - jc * _L
                for m in range(n_mas):
                    a = m % _NACC
                    v = plsc.load_gather(ptb, [bvec, rvec, idx16 + m * n_ops])
                    u = plsc.bitcast(v, jnp.uint32) - jnp.uint32(1)
                    if pad < _L:
                        u = jnp.where(lane < pad, u, umax)
                    better = u < uvals[a]
                    uvals[a] = jnp.where(better, u, uvals[a])
                    keys[a] = jnp.where(better, jkey + m, keys[a])
            vm, km = uvals[0], keys[0]
            for a in range(1, _NACC):
                take = (uvals[a] < vm) | ((uvals[a] == vm) & (keys[a] < km))
                vm = jnp.where(take, uvals[a], vm)
                km = jnp.where(take, keys[a], km)
            sm = plsc.bitcast(vm ^ jnp.uint32(0x80000000), jnp.int32)
            minv = jnp.min(sm)
            fkey = jnp.min(jnp.where(sm == minv, km, _IBIG))
            fkey = jnp.where(minv == jnp.int32(0x7FFFFFFF), 0, fkey)
            tv = tbub[row]
            tkey = jnp.min(jnp.where(tv == jnp.min(tv), lane, _L))
            act = 1 + fkey * n_trs + tkey
            actb[r] = jnp.full((_L,), act, jnp.int32)
            return carry2

        lax.fori_loop(0, _RPC, row_body, 0)
        pltpu.sync_copy(actb, out_ref.at[pl.ds(base + c * _RPC, _RPC)])

        @pl.when(c + 2 < nchunk)
        def _prefetch():
            pltpu.async_copy(pt_ref.at[pl.ds(base + (c + 2) * _RPC, _RPC)],
                             ptb.at[buf], sem)

        return carry

    lax.fori_loop(0, nchunk, chunk_body, 0)


def _tc_onehot(act_ref, out_ref):
    act = act_ref[:, :1]
    n_cols = out_ref.shape[1]
    col = lax.broadcasted_iota(jnp.int32, (act_ref.shape[0], n_cols), 1)
    out_ref[...] = jnp.where(col == act, 1.0, 0.0).astype(jnp.float32)


def kernel(job_done, machine_busy_until, truck_location, next_op, proc_times,
           truck_busy_until, action_mask):
    B, n_jobs = job_done.shape
    n_mas = machine_busy_until.shape[1]
    n_trs = truck_location.shape[1]
    n_ops = proc_times.shape[2]
    n_cols = 1 + n_jobs * n_mas * n_trs
    rows = B // _NW

    jpad = (-n_jobs) % _L
    nop_p = jnp.pad(next_op, ((0, 0), (0, jpad)))
    tbu_p = jnp.pad(truck_busy_until, ((0, 0), (0, _L - n_trs)),
                    constant_values=jnp.inf)
    pt2 = proc_times.reshape(B, n_mas * n_ops)

    sel = functools.partial(_sc_select, rows=rows, n_jobs=n_jobs,
                            n_mas=n_mas, n_trs=n_trs, n_ops=n_ops)
    act16 = pl.kernel(
        sel,
        out_type=jax.ShapeDtypeStruct((B, _L), jnp.int32),
        mesh=plsc.VectorSubcoreMesh(core_axis_name="c", subcore_axis_name="s",
                                    num_cores=_NC, num_subcores=_NS),
        compiler_params=pltpu.CompilerParams(needs_layout_passes=False),
        scratch_types=[
            pltpu.VMEM((2, _RPC, n_mas * n_ops), jnp.float32),
            pltpu.VMEM((rows, n_jobs + jpad), jnp.int32),
            pltpu.VMEM((rows, _L), jnp.float32),
            pltpu.VMEM((_RPC, _L), jnp.int32),
            pltpu.SemaphoreType.DMA,
        ],
    )(nop_p, pt2, tbu_p)

    logits = pl.pallas_call(
        _tc_onehot,
        grid=(B // _BB,),
        in_specs=[pl.BlockSpec((_BB, _L), lambda i: (i, 0))],
        out_specs=pl.BlockSpec((_BB, n_cols), lambda i: (i, 0)),
        out_shape=jax.ShapeDtypeStruct((B, n_cols), jnp.float32),
    )(act16)
    return (logits, action_mask)


# speedup vs baseline: 1.0491x; 1.0196x over previous
import functools

import jax
import jax.numpy as jnp
from jax import lax
from jax.experimental import pallas as pl
from jax.experimental.pallas import tpu as pltpu
from jax.experimental.pallas import tpu_sc as plsc

_IBIG = 1 << 20
_NC, _NS, _L = 2, 16, 16
_NW = _NC * _NS
_RPC = 8
_NACC = 8
_BB = 256


def _sc_select(nop_ref, pt_ref, tbu_ref, out_ref, ptb, nopb, tbub, actb, sem,
               *, rows, n_jobs, n_mas, n_trs, n_ops, offset):
    cid = lax.axis_index("c")
    sid = lax.axis_index("s")
    wid = sid * _NC + cid
    obase = wid * rows
    base = offset + obase
    n_jc = nopb.shape[1] // _L
    nchunk = rows // _RPC
    lane = lax.iota(jnp.int32, _L)

    pltpu.sync_copy(nop_ref.at[pl.ds(base, rows)], nopb)
    pltpu.sync_copy(tbu_ref.at[pl.ds(base, rows)], tbub)
    pltpu.async_copy(pt_ref.at[pl.ds(base, _RPC)], ptb.at[0], sem)
    pltpu.async_copy(pt_ref.at[pl.ds(base + _RPC, _RPC)], ptb.at[1], sem)

    def chunk_body(c, carry):
        buf = lax.rem(c, 2)
        pltpu.make_async_copy(pt_ref.at[pl.ds(base, _RPC)], ptb.at[0], sem).wait()
        bvec = jnp.full((_L,), buf, jnp.int32)

        def row_body(r, carry2):
            row = c * _RPC + r
            rvec = jnp.full((_L,), r, jnp.int32)
            umax = jnp.uint32(0xFFFFFFFF)
            uvals = [jnp.full((_L,), umax, jnp.uint32) for _ in range(_NACC)]
            keys = [jnp.full((_L,), _IBIG, jnp.int32) for _ in range(_NACC)]
            for jc in range(n_jc):
                idx16 = nopb[row, pl.ds(jc * _L, _L)]
                jkey = (jc * _L + lane) * n_mas
                pad = n_jobs - jc * _L
                for m in range(n_mas):
                    a = m % _NACC
                    v = plsc.load_gather(ptb, [bvec, rvec, idx16 + m * n_ops])
                    u = plsc.bitcast(v, jnp.uint32) - jnp.uint32(1)
                    if pad < _L:
                        u = jnp.where(lane < pad, u, umax)
                    better = u < uvals[a]
                    uvals[a] = jnp.where(better, u, uvals[a])
                    keys[a] = jnp.where(better, jkey + m, keys[a])
            vm, km = uvals[0], keys[0]
            for a in range(1, _NACC):
                take = (uvals[a] < vm) | ((uvals[a] == vm) & (keys[a] < km))
                vm = jnp.where(take, uvals[a], vm)
                km = jnp.where(take, keys[a], km)
            sm = plsc.bitcast(vm ^ jnp.uint32(0x80000000), jnp.int32)
            minv = jnp.min(sm)
            fkey = jnp.min(jnp.where(sm == minv, km, _IBIG))
            fkey = jnp.where(minv == jnp.int32(0x7FFFFFFF), 0, fkey)
            tv = tbub[row]
            tkey = jnp.min(jnp.where(tv == jnp.min(tv), lane, _L))
            act = 1 + fkey * n_trs + tkey
            actb[r] = jnp.full((_L,), act, jnp.int32)
            return carry2

        lax.fori_loop(0, _RPC, row_body, 0)
        pltpu.sync_copy(actb, out_ref.at[pl.ds(obase + c * _RPC, _RPC)])

        @pl.when(c + 2 < nchunk)
        def _prefetch():
            pltpu.async_copy(pt_ref.at[pl.ds(base + (c + 2) * _RPC, _RPC)],
                             ptb.at[buf], sem)

        return carry

    lax.fori_loop(0, nchunk, chunk_body, 0)


def _tc_onehot(act_ref, out_ref):
    act = act_ref[:, :1]
    n_cols = out_ref.shape[1]
    col = lax.broadcasted_iota(jnp.int32, (act_ref.shape[0], n_cols), 1)
    out_ref[...] = jnp.where(col == act, 1.0, 0.0).astype(jnp.float32)


def _tc_onehot_alias(act_ref, prev_ref, out_ref):
    del prev_ref
    _tc_onehot(act_ref, out_ref)


def kernel(job_done, machine_busy_until, truck_location, next_op, proc_times,
           truck_busy_until, action_mask):
    B, n_jobs = job_done.shape
    n_mas = machine_busy_until.shape[1]
    n_trs = truck_location.shape[1]
    n_ops = proc_times.shape[2]
    n_cols = 1 + n_jobs * n_mas * n_trs
    nsplit = 2
    Bh = B // nsplit
    rows = Bh // _NW

    jpad = (-n_jobs) % _L
    nop_p = jnp.pad(next_op, ((0, 0), (0, jpad)))
    tbu_p = jnp.pad(truck_busy_until, ((0, 0), (0, _L - n_trs)),
                    constant_values=jnp.inf)
    pt2 = proc_times.reshape(B, n_mas * n_ops)

    def sc_half(offset):
        sel = functools.partial(_sc_select, rows=rows, n_jobs=n_jobs,
                                n_mas=n_mas, n_trs=n_trs, n_ops=n_ops,
                                offset=offset)
        return pl.kernel(
            sel,
            out_type=jax.ShapeDtypeStruct((Bh, _L), jnp.int32),
            mesh=plsc.VectorSubcoreMesh(core_axis_name="c",
                                        subcore_axis_name="s",
                                        num_cores=_NC, num_subcores=_NS),
            compiler_params=pltpu.CompilerParams(needs_layout_passes=False),
            scratch_types=[
                pltpu.VMEM((2, _RPC, n_mas * n_ops), jnp.float32),
                pltpu.VMEM((rows, n_jobs + jpad), jnp.int32),
                pltpu.VMEM((rows, _L), jnp.float32),
                pltpu.VMEM((_RPC, _L), jnp.int32),
                pltpu.SemaphoreType.DMA,
            ],
        )(nop_p, pt2, tbu_p)

    acts = [sc_half(i * Bh) for i in range(nsplit)]

    nblk = Bh // _BB
    logits = pl.pallas_call(
        _tc_onehot,
        grid=(nblk,),
        in_specs=[pl.BlockSpec((_BB, _L), lambda i: (i, 0))],
        out_specs=pl.BlockSpec((_BB, n_cols), lambda i: (i, 0)),
        out_shape=jax.ShapeDtypeStruct((B, n_cols), jnp.float32),
    )(acts[0])
    for h in range(1, nsplit):
        logits = pl.pallas_call(
            _tc_onehot_alias,
            grid=(nblk,),
            in_specs=[pl.BlockSpec((_BB, _L), lambda i: (i, 0)),
                      pl.BlockSpec(memory_space=pl.ANY)],
            out_specs=pl.BlockSpec((_BB, n_cols),
                                   lambda i, h=h: (i + h * nblk, 0)),
            out_shape=jax.ShapeDtypeStruct((B, n_cols), jnp.float32),
            input_output_aliases={1: 0},
        )(acts[h], logits)
    return (logits, action_mask)
